# Initial kernel scaffold; baseline (speedup 1.0000x reference)
import jax
import jax.numpy as jnp
from jax.experimental import pallas as pl


def kernel(s, t, edge_index, W_sm1, b_sm1, W_sm4, b_sm4, W_sl1, b_sl1, W_sl4, b_sl4, W_tm1, b_tm1, W_tm4, b_tm4, W_tl1, b_tl1, W_tl4, b_tl4):
    raise NotImplementedError("write your pallas kernel here")


# speedup vs baseline: 9.1499x; 9.1499x over previous
import functools

import jax
import jax.numpy as jnp
from jax import lax
from jax.experimental import pallas as pl
from jax.experimental.pallas import tpu as pltpu
from jax.experimental.pallas import tpu_sc as plsc

N = 10000
D = 128
E = 320000

NC = 2
NS = 16
CHUNK = 80
EPT = E // NS
NCH = EPT // CHUNK
ZROWS = 640

RB = 1000
NB = N // RB

_mesh = plsc.VectorSubcoreMesh(
    core_axis_name="c", subcore_axis_name="s", num_cores=NC, num_subcores=NS
)


@functools.partial(
    pl.kernel,
    out_type=jax.ShapeDtypeStruct((2 * N,), jnp.float32),
    mesh=_mesh,
    scratch_types=[
        pltpu.VMEM((CHUNK,), jnp.int32),
        pltpu.VMEM((CHUNK,), jnp.float32),
        pltpu.VMEM((ZROWS,), jnp.float32),
        pltpu.VMEM_SHARED((N,), jnp.float32),
    ],
)
def _deg_kernel(ei_ref, out_ref, idx_v, ones_v, zbuf_v, acc_sh):
    c = lax.axis_index("c")
    s = lax.axis_index("s")

    for j in range(CHUNK // 16):
        ones_v[pl.ds(16 * j, 16)] = jnp.ones((16,), jnp.float32)
    for j in range(ZROWS // 16):
        zbuf_v[pl.ds(16 * j, 16)] = jnp.zeros((16,), jnp.float32)

    @pl.when(s < NS - 1)
    def _():
        pltpu.sync_copy(zbuf_v, acc_sh.at[pl.ds(s * ZROWS, ZROWS)])

    @pl.when(s == NS - 1)
    def _():
        pltpu.sync_copy(zbuf_v.at[pl.ds(0, 400)], acc_sh.at[pl.ds(9600, 400)])

    plsc.subcore_barrier()

    def body(k, carry):
        base = c * E + s * EPT + k * CHUNK
        pltpu.sync_copy(ei_ref.at[pl.ds(base, CHUNK)], idx_v)
        pltpu.sync_copy(ones_v, acc_sh.at[idx_v], add=True)
        return carry

    lax.fori_loop(0, NCH, body, 0)
    plsc.subcore_barrier()

    @pl.when(s < NS - 1)
    def _():
        pltpu.sync_copy(acc_sh.at[pl.ds(s * ZROWS, ZROWS)], zbuf_v)
        pltpu.sync_copy(zbuf_v, out_ref.at[pl.ds(c * N + s * ZROWS, ZROWS)])

    @pl.when(s == NS - 1)
    def _():
        pltpu.sync_copy(acc_sh.at[pl.ds(9600, 400)], zbuf_v.at[pl.ds(0, 400)])
        pltpu.sync_copy(zbuf_v.at[pl.ds(0, 400)], out_ref.at[pl.ds(c * N + 9600, 400)])


def _make_agg(swap):
    @functools.partial(
        pl.kernel,
        out_type=jax.ShapeDtypeStruct((2 * N, D), jnp.float32),
        mesh=_mesh,
        scratch_types=[
            pltpu.VMEM((CHUNK,), jnp.int32),
            pltpu.VMEM((CHUNK,), jnp.int32),
            pltpu.VMEM((CHUNK,), jnp.int32),
            pltpu.VMEM((CHUNK, D), jnp.float32),
            pltpu.VMEM_SHARED((N, D), jnp.float32),
            pltpu.SemaphoreType.DMA,
        ],
    )
    def _agg(ei_ref, h_ref, out_ref, gidx_v, gidx2_v, sidx_v, rows_v, acc_sh, sem):
        c = lax.axis_index("c")
        s = lax.axis_index("s")
        coff = c * N
        grow = (1 - c) if swap else c

        def zb(r, carry):
            for j in range(D // 16):
                rows_v[r, pl.ds(16 * j, 16)] = jnp.zeros((16,), jnp.float32)
            return carry

        lax.fori_loop(0, CHUNK, zb, 0)

        for j in range(ZROWS // CHUNK):
            r0 = s * ZROWS + j * CHUNK

            def cp(r0=r0):
                pltpu.sync_copy(rows_v, acc_sh.at[pl.ds(r0, CHUNK)])

            if j < 5:
                cp()
            else:
                pl.when(s < NS - 1)(cp)

        plsc.subcore_barrier()

        def body(k, carry):
            base = s * EPT + k * CHUNK
            goff = grow * E + base
            soff = (1 - grow) * E + base
            pltpu.sync_copy(ei_ref.at[pl.ds(goff, CHUNK)], gidx_v)
            pltpu.sync_copy(ei_ref.at[pl.ds(soff, CHUNK)], sidx_v)
            for j in range(CHUNK // 16):
                gidx2_v[pl.ds(16 * j, 16)] = gidx_v[pl.ds(16 * j, 16)] + coff
            pltpu.async_copy(h_ref.at[gidx2_v], rows_v, sem).wait()
            pltpu.sync_copy(rows_v, acc_sh.at[sidx_v], add=True)
            return carry

        lax.fori_loop(0, NCH, body, 0)
        plsc.subcore_barrier()

        for j in range(ZROWS // CHUNK):
            r0 = s * ZROWS + j * CHUNK

            def wb(r0=r0):
                pltpu.sync_copy(acc_sh.at[pl.ds(r0, CHUNK)], rows_v)
                pltpu.sync_copy(rows_v, out_ref.at[pl.ds(coff + r0, CHUNK)])

            if j < 5:
                wb()
            else:
                pl.when(s < NS - 1)(wb)

    return _agg


_agg_a = _make_agg(False)
_agg_b = _make_agg(True)


def _rscales(deg_blk):
    r_out = lax.rsqrt(deg_blk[:, 0:1] + 1.0)
    r_in = lax.rsqrt(deg_blk[:, 1:2] + 1.0)
    return r_out, r_in


def _mm(x, w_ref, b_ref):
    w = w_ref[0]
    b = b_ref[0]
    return (
        lax.dot_general(x, w, (((1,), (1,)), ((), ())),
                        preferred_element_type=jnp.float32)
        + b
    )


def _tc1_body(deg_ref, x_ref, w_ref, b_ref, hsc_ref, diag_ref):
    chain = pl.program_id(0)
    r_out, r_in = _rscales(deg_ref[...])
    h = _mm(x_ref[...], w_ref, b_ref)
    f = jnp.where(chain == 0, r_out, r_in)
    hsc_ref[...] = h * f
    diag_ref[...] = h * (r_in * r_out)


def _tc2_body(deg_ref, raw_ref, diag_ref, w_ref, b_ref, zsc_ref, diag2_ref):
    chain = pl.program_id(0)
    r_out, r_in = _rscales(deg_ref[...])
    f1 = jnp.where(chain == 0, r_in, r_out)
    a = jnp.maximum(raw_ref[...] * f1 + diag_ref[...], 0.0)
    z = _mm(a, w_ref, b_ref)
    zsc_ref[...] = z * f1
    diag2_ref[...] = z * (r_in * r_out)


def _tc3_body(deg_ref, raw_ref, diag2_ref, mu_ref):
    chain = pl.program_id(0)
    r_out, r_in = _rscales(deg_ref[...])
    f2 = jnp.where(chain == 0, r_out, r_in)
    mu_ref[...] = raw_ref[...] * f2 + diag2_ref[...]


def _row_spec():
    return pl.BlockSpec((RB, D), lambda c, i: (c * NB + i, 0))


def _deg_spec():
    return pl.BlockSpec((RB, 2), lambda c, i: (i, 0))


def _w_spec():
    return pl.BlockSpec((1, D, D), lambda c, i: (c, 0, 0))


def _b_spec():
    return pl.BlockSpec((1, 1, D), lambda c, i: (c, 0, 0))


def _tc1(degT, xs, w1, b1):
    return pl.pallas_call(
        _tc1_body,
        grid=(2, NB),
        in_specs=[_deg_spec(), _row_spec(), _w_spec(), _b_spec()],
        out_specs=[_row_spec(), _row_spec()],
        out_shape=[jax.ShapeDtypeStruct((2 * N, D), jnp.float32)] * 2,
    )(degT, xs, w1, b1)


def _tc2(degT, raw, diag, w4, b4):
    return pl.pallas_call(
        _tc2_body,
        grid=(2, NB),
        in_specs=[_deg_spec(), _row_spec(), _row_spec(), _w_spec(), _b_spec()],
        out_specs=[_row_spec(), _row_spec()],
        out_shape=[jax.ShapeDtypeStruct((2 * N, D), jnp.float32)] * 2,
    )(degT, raw, diag, w4, b4)


def _tc3(degT, raw, diag2):
    return pl.pallas_call(
        _tc3_body,
        grid=(2, NB),
        in_specs=[_deg_spec(), _row_spec(), _row_spec()],
        out_specs=_row_spec(),
        out_shape=jax.ShapeDtypeStruct((2 * N, D), jnp.float32),
    )(degT, raw, diag2)


def kernel(s, t, edge_index, W_sm1, b_sm1, W_sm4, b_sm4, W_sl1, b_sl1,
           W_sl4, b_sl4, W_tm1, b_tm1, W_tm4, b_tm4, W_tl1, b_tl1,
           W_tl4, b_tl4):
    ei_flat = edge_index.reshape(2 * E)
    deg = _deg_kernel(ei_flat)
    degT = deg.reshape(2, N).T

    xs = jnp.concatenate([s, t], axis=0)
    w1 = jnp.stack([W_sm1, W_tm1])
    b1 = jnp.stack([b_sm1, b_tm1])[:, None, :]
    w4 = jnp.stack([W_sm4, W_tm4])
    b4 = jnp.stack([b_sm4, b_tm4])[:, None, :]

    hsc, diag1 = _tc1(degT, xs, w1, b1)
    raw_a = _agg_a(ei_flat, hsc)
    zsc, diag2 = _tc2(degT, raw_a, diag1, w4, b4)
    raw_b = _agg_b(ei_flat, zsc)
    mu = _tc3(degT, raw_b, diag2)
    return mu[:N], mu[N:]


# speedup vs baseline: 17.2308x; 1.8832x over previous
import functools

import jax
import jax.numpy as jnp
from jax import lax
from jax.experimental import pallas as pl
from jax.experimental.pallas import tpu as pltpu
from jax.experimental.pallas import tpu_sc as plsc

N = 10000
D = 128
E = 320000

NC = 2
NS = 16
CHUNK = 80
EPT = E // NS
NCH = EPT // CHUNK
NCH2 = NCH // 2
ZROWS = 640

RB = 1000
NB = N // RB

_mesh = plsc.VectorSubcoreMesh(
    core_axis_name="c", subcore_axis_name="s", num_cores=NC, num_subcores=NS
)


@functools.partial(
    pl.kernel,
    out_type=jax.ShapeDtypeStruct((2 * N,), jnp.float32),
    mesh=_mesh,
    scratch_types=[
        pltpu.VMEM((CHUNK,), jnp.int32),
        pltpu.VMEM((CHUNK,), jnp.int32),
        pltpu.VMEM((CHUNK,), jnp.float32),
        pltpu.VMEM((ZROWS,), jnp.float32),
        pltpu.VMEM_SHARED((N,), jnp.float32),
        pltpu.SemaphoreType.DMA,
        pltpu.SemaphoreType.DMA,
    ],
)
def _deg_kernel(ei_ref, out_ref, idx0_v, idx1_v, ones_v, zbuf_v, acc_sh,
                isem0, isem1):
    c = lax.axis_index("c")
    s = lax.axis_index("s")

    for j in range(CHUNK // 16):
        ones_v[pl.ds(16 * j, 16)] = jnp.ones((16,), jnp.float32)
    for j in range(ZROWS // 16):
        zbuf_v[pl.ds(16 * j, 16)] = jnp.zeros((16,), jnp.float32)

    @pl.when(s < NS - 1)
    def _():
        pltpu.sync_copy(zbuf_v, acc_sh.at[pl.ds(s * ZROWS, ZROWS)])

    @pl.when(s == NS - 1)
    def _():
        pltpu.sync_copy(zbuf_v.at[pl.ds(0, 400)], acc_sh.at[pl.ds(9600, 400)])

    plsc.subcore_barrier()

    base0 = c * E + s * EPT

    def istart(k, buf, sem):
        pltpu.async_copy(ei_ref.at[pl.ds(base0 + k * CHUNK, CHUNK)], buf, sem)

    def iwait(buf, sem):
        pltpu.make_async_copy(ei_ref.at[pl.ds(0, CHUNK)], buf, sem).wait()

    istart(0, idx0_v, isem0)

    def body(kk, carry):
        k0 = kk * 2
        k1 = k0 + 1
        istart(k1, idx1_v, isem1)
        iwait(idx0_v, isem0)
        pltpu.sync_copy(ones_v, acc_sh.at[idx0_v], add=True)

        @pl.when(kk + 1 < NCH2)
        def _():
            istart(k0 + 2, idx0_v, isem0)

        iwait(idx1_v, isem1)
        pltpu.sync_copy(ones_v, acc_sh.at[idx1_v], add=True)
        return carry

    lax.fori_loop(0, NCH2, body, 0)
    plsc.subcore_barrier()

    @pl.when(s < NS - 1)
    def _():
        pltpu.sync_copy(acc_sh.at[pl.ds(s * ZROWS, ZROWS)], zbuf_v)
        pltpu.sync_copy(zbuf_v, out_ref.at[pl.ds(c * N + s * ZROWS, ZROWS)])

    @pl.when(s == NS - 1)
    def _():
        pltpu.sync_copy(acc_sh.at[pl.ds(9600, 400)], zbuf_v.at[pl.ds(0, 400)])
        pltpu.sync_copy(zbuf_v.at[pl.ds(0, 400)], out_ref.at[pl.ds(c * N + 9600, 400)])


def _make_agg(swap):
    @functools.partial(
        pl.kernel,
        out_type=jax.ShapeDtypeStruct((2 * N, D), jnp.float32),
        mesh=_mesh,
        scratch_types=[
            pltpu.VMEM((EPT,), jnp.int32),
            pltpu.VMEM((CHUNK,), jnp.int32),
            pltpu.VMEM((CHUNK,), jnp.int32),
            pltpu.VMEM((CHUNK, D), jnp.float32),
            pltpu.VMEM((CHUNK, D), jnp.float32),
            pltpu.VMEM_SHARED((N, D), jnp.float32),
            pltpu.SemaphoreType.DMA,
            pltpu.SemaphoreType.DMA,
            pltpu.SemaphoreType.DMA,
            pltpu.SemaphoreType.DMA,
        ],
    )
    def _agg(ei_ref, h_ref, out_ref, gidx_flat, sidx0, sidx1, rows0, rows1,
             acc_sh, gsem0, gsem1, isem0, isem1):
        c = lax.axis_index("c")
        s = lax.axis_index("s")
        coff = c * N
        grow = (1 - c) if swap else c

        pltpu.sync_copy(ei_ref.at[pl.ds(grow * E + s * EPT, EPT)], gidx_flat)

        def adj(i, carry):
            gidx_flat[pl.ds(i * 16, 16)] = gidx_flat[pl.ds(i * 16, 16)] + coff
            return carry

        lax.fori_loop(0, EPT // 16, adj, 0)

        def zb(r, carry):
            for j in range(D // 16):
                rows0[r, pl.ds(16 * j, 16)] = jnp.zeros((16,), jnp.float32)
            return carry

        lax.fori_loop(0, CHUNK, zb, 0)

        for j in range(ZROWS // CHUNK):
            r0 = s * ZROWS + j * CHUNK

            def cp(r0=r0):
                pltpu.sync_copy(rows0, acc_sh.at[pl.ds(r0, CHUNK)])

            if j < 5:
                cp()
            else:
                pl.when(s < NS - 1)(cp)

        plsc.subcore_barrier()

        sbase = (1 - grow) * E + s * EPT

        def istart(k, buf, sem):
            pltpu.async_copy(ei_ref.at[pl.ds(sbase + k * CHUNK, CHUNK)], buf,
                             sem)

        def iwait(buf, sem):
            pltpu.make_async_copy(ei_ref.at[pl.ds(0, CHUNK)], buf, sem).wait()

        def gstart(k, buf, sem):
            pltpu.async_copy(
                h_ref.at[gidx_flat.at[pl.ds(k * CHUNK, CHUNK)]], buf, sem)

        def gwait(buf, sem):
            pltpu.make_async_copy(h_ref.at[pl.ds(0, CHUNK)], buf, sem).wait()

        istart(0, sidx0, isem0)
        gstart(0, rows0, gsem0)

        def body(kk, carry):
            k0 = kk * 2
            k1 = k0 + 1
            istart(k1, sidx1, isem1)
            gwait(rows0, gsem0)
            gstart(k1, rows1, gsem1)
            iwait(sidx0, isem0)
            pltpu.sync_copy(rows0, acc_sh.at[sidx0], add=True)

            @pl.when(kk + 1 < NCH2)
            def _():
                istart(k0 + 2, sidx0, isem0)

            gwait(rows1, gsem1)

            @pl.when(kk + 1 < NCH2)
            def _():
                gstart(k0 + 2, rows0, gsem0)

            iwait(sidx1, isem1)
            pltpu.sync_copy(rows1, acc_sh.at[sidx1], add=True)
            return carry

        lax.fori_loop(0, NCH2, body, 0)
        plsc.subcore_barrier()

        for j in range(ZROWS // CHUNK):
            r0 = s * ZROWS + j * CHUNK

            def wb(r0=r0):
                pltpu.sync_copy(acc_sh.at[pl.ds(r0, CHUNK)], rows0)
                pltpu.sync_copy(rows0, out_ref.at[pl.ds(coff + r0, CHUNK)])

            if j < 5:
                wb()
            else:
                pl.when(s < NS - 1)(wb)

    return _agg


_agg_a = _make_agg(False)
_agg_b = _make_agg(True)


def _rscales(deg_blk):
    r_out = lax.rsqrt(deg_blk[:, 0:1] + 1.0)
    r_in = lax.rsqrt(deg_blk[:, 1:2] + 1.0)
    return r_out, r_in


def _mm(x, w_ref, b_ref):
    w = w_ref[0]
    b = b_ref[0]
    return (
        lax.dot_general(x, w, (((1,), (1,)), ((), ())),
                        preferred_element_type=jnp.float32)
        + b
    )


def _tc1_body(deg_ref, x_ref, w_ref, b_ref, hsc_ref, diag_ref):
    chain = pl.program_id(0)
    r_out, r_in = _rscales(deg_ref[...])
    h = _mm(x_ref[...], w_ref, b_ref)
    f = jnp.where(chain == 0, r_out, r_in)
    hsc_ref[...] = h * f
    diag_ref[...] = h * (r_in * r_out)


def _tc2_body(deg_ref, raw_ref, diag_ref, w_ref, b_ref, zsc_ref, diag2_ref):
    chain = pl.program_id(0)
    r_out, r_in = _rscales(deg_ref[...])
    f1 = jnp.where(chain == 0, r_in, r_out)
    a = jnp.maximum(raw_ref[...] * f1 + diag_ref[...], 0.0)
    z = _mm(a, w_ref, b_ref)
    zsc_ref[...] = z * f1
    diag2_ref[...] = z * (r_in * r_out)


def _tc3_body(deg_ref, raw_ref, diag2_ref, mu_ref):
    chain = pl.program_id(0)
    r_out, r_in = _rscales(deg_ref[...])
    f2 = jnp.where(chain == 0, r_out, r_in)
    mu_ref[...] = raw_ref[...] * f2 + diag2_ref[...]


def _row_spec():
    return pl.BlockSpec((RB, D), lambda c, i: (c * NB + i, 0))


def _deg_spec():
    return pl.BlockSpec((RB, 2), lambda c, i: (i, 0))


def _w_spec():
    return pl.BlockSpec((1, D, D), lambda c, i: (c, 0, 0))


def _b_spec():
    return pl.BlockSpec((1, 1, D), lambda c, i: (c, 0, 0))


def _tc1(degT, xs, w1, b1):
    return pl.pallas_call(
        _tc1_body,
        grid=(2, NB),
        in_specs=[_deg_spec(), _row_spec(), _w_spec(), _b_spec()],
        out_specs=[_row_spec(), _row_spec()],
        out_shape=[jax.ShapeDtypeStruct((2 * N, D), jnp.float32)] * 2,
    )(degT, xs, w1, b1)


def _tc2(degT, raw, diag, w4, b4):
    return pl.pallas_call(
        _tc2_body,
        grid=(2, NB),
        in_specs=[_deg_spec(), _row_spec(), _row_spec(), _w_spec(), _b_spec()],
        out_specs=[_row_spec(), _row_spec()],
        out_shape=[jax.ShapeDtypeStruct((2 * N, D), jnp.float32)] * 2,
    )(degT, raw, diag, w4, b4)


def _tc3(degT, raw, diag2):
    return pl.pallas_call(
        _tc3_body,
        grid=(2, NB),
        in_specs=[_deg_spec(), _row_spec(), _row_spec()],
        out_specs=_row_spec(),
        out_shape=jax.ShapeDtypeStruct((2 * N, D), jnp.float32),
    )(degT, raw, diag2)


def kernel(s, t, edge_index, W_sm1, b_sm1, W_sm4, b_sm4, W_sl1, b_sl1,
           W_sl4, b_sl4, W_tm1, b_tm1, W_tm4, b_tm4, W_tl1, b_tl1,
           W_tl4, b_tl4):
    ei_flat = edge_index.reshape(2 * E)
    deg = _deg_kernel(ei_flat)
    degT = deg.reshape(2, N).T

    xs = jnp.concatenate([s, t], axis=0)
    w1 = jnp.stack([W_sm1, W_tm1])
    b1 = jnp.stack([b_sm1, b_tm1])[:, None, :]
    w4 = jnp.stack([W_sm4, W_tm4])
    b4 = jnp.stack([b_sm4, b_tm4])[:, None, :]

    hsc, diag1 = _tc1(degT, xs, w1, b1)
    raw_a = _agg_a(ei_flat, hsc)
    zsc, diag2 = _tc2(degT, raw_a, diag1, w4, b4)
    raw_b = _agg_b(ei_flat, zsc)
    mu = _tc3(degT, raw_b, diag2)
    return mu[:N], mu[N:]


# speedup vs baseline: 25.8191x; 1.4984x over previous
import functools

import jax
import jax.numpy as jnp
from jax import lax
from jax.experimental import pallas as pl
from jax.experimental.pallas import tpu as pltpu
from jax.experimental.pallas import tpu_sc as plsc

N = 10000
D = 128
E = 320000

NC = 2
NS = 16
CHUNK = 80
RING = 2
EPT = E // NS
NCH = EPT // CHUNK
ZROWS = 640

RB = 1000
NB = N // RB

_mesh = plsc.VectorSubcoreMesh(
    core_axis_name="c", subcore_axis_name="s", num_cores=NC, num_subcores=NS
)


@functools.partial(
    pl.kernel,
    out_type=jax.ShapeDtypeStruct((2 * N,), jnp.float32),
    mesh=_mesh,
    scratch_types=(
        [pltpu.VMEM((CHUNK,), jnp.int32)] * RING
        + [pltpu.VMEM((CHUNK,), jnp.float32),
           pltpu.VMEM((ZROWS,), jnp.float32),
           pltpu.VMEM_SHARED((N,), jnp.float32)]
        + [pltpu.SemaphoreType.DMA] * RING
    ),
)
def _deg_kernel(ei_ref, out_ref, *scr):
    ibufs = scr[:RING]
    ones_v, zbuf_v, acc_sh = scr[RING:RING + 3]
    isems = scr[RING + 3:]
    c = lax.axis_index("c")
    s = lax.axis_index("s")

    for j in range(CHUNK // 16):
        ones_v[pl.ds(16 * j, 16)] = jnp.ones((16,), jnp.float32)
    for j in range(ZROWS // 16):
        zbuf_v[pl.ds(16 * j, 16)] = jnp.zeros((16,), jnp.float32)

    @pl.when(s < NS - 1)
    def _():
        pltpu.sync_copy(zbuf_v, acc_sh.at[pl.ds(s * ZROWS, ZROWS)])

    @pl.when(s == NS - 1)
    def _():
        pltpu.sync_copy(zbuf_v.at[pl.ds(0, 400)], acc_sh.at[pl.ds(9600, 400)])

    plsc.subcore_barrier()

    base0 = c * E + s * EPT

    def istart(k, buf, sem):
        pltpu.async_copy(ei_ref.at[pl.ds(base0 + k * CHUNK, CHUNK)], buf, sem)

    def iwait(buf, sem):
        pltpu.make_async_copy(ei_ref.at[pl.ds(0, CHUNK)], buf, sem).wait()

    for b in range(RING):
        istart(b, ibufs[b], isems[b])

    def body(kk, carry):
        for b in range(RING):
            k = kk * RING + b
            iwait(ibufs[b], isems[b])
            pltpu.sync_copy(ones_v, acc_sh.at[ibufs[b]], add=True)

            @pl.when(k + RING < NCH)
            def _(b=b, k=k):
                istart(k + RING, ibufs[b], isems[b])

        return carry

    lax.fori_loop(0, NCH // RING, body, 0)
    for k in range((NCH // RING) * RING, NCH):
        b = k % RING
        iwait(ibufs[b], isems[b])
        pltpu.sync_copy(ones_v, acc_sh.at[ibufs[b]], add=True)
    plsc.subcore_barrier()

    @pl.when(s < NS - 1)
    def _():
        pltpu.sync_copy(acc_sh.at[pl.ds(s * ZROWS, ZROWS)], zbuf_v)
        pltpu.sync_copy(zbuf_v, out_ref.at[pl.ds(c * N + s * ZROWS, ZROWS)])

    @pl.when(s == NS - 1)
    def _():
        pltpu.sync_copy(acc_sh.at[pl.ds(9600, 400)], zbuf_v.at[pl.ds(0, 400)])
        pltpu.sync_copy(zbuf_v.at[pl.ds(0, 400)], out_ref.at[pl.ds(c * N + 9600, 400)])


def _make_agg(swap):
    @functools.partial(
        pl.kernel,
        out_type=jax.ShapeDtypeStruct((2 * N, D), jnp.float32),
        mesh=_mesh,
        scratch_types=(
            [pltpu.VMEM((EPT // 2,), jnp.int32)]
            + [pltpu.VMEM((CHUNK,), jnp.int32)] * RING
            + [pltpu.VMEM((CHUNK, D), jnp.float32)] * RING
            + [pltpu.VMEM_SHARED((N, D), jnp.float32)]
            + [pltpu.SemaphoreType.DMA] * (2 * RING)
        ),
    )
    def _agg(ei_ref, h_ref, out_ref, *scr):
        gidx_flat = scr[0]
        sbufs = scr[1:1 + RING]
        rbufs = scr[1 + RING:1 + 2 * RING]
        acc_sh = scr[1 + 2 * RING]
        gsems = scr[2 + 2 * RING:2 + 3 * RING]
        isems = scr[2 + 3 * RING:]
        rows0 = rbufs[0]
        c = lax.axis_index("c")
        s = lax.axis_index("s")
        coff = c * N
        grow = (1 - c) if swap else c

        def load_gidx(ph):
            pltpu.sync_copy(
                ei_ref.at[pl.ds(grow * E + s * EPT + ph * (EPT // 2),
                                EPT // 2)],
                gidx_flat,
            )

            def adj(i, carry):
                gidx_flat[pl.ds(i * 16, 16)] = (
                    gidx_flat[pl.ds(i * 16, 16)] + coff
                )
                return carry

            lax.fori_loop(0, EPT // 32, adj, 0)

        def zb(r, carry):
            for j in range(D // 16):
                rows0[r, pl.ds(16 * j, 16)] = jnp.zeros((16,), jnp.float32)
            return carry

        lax.fori_loop(0, CHUNK, zb, 0)

        for j in range(ZROWS // CHUNK):
            r0 = s * ZROWS + j * CHUNK

            def cp(r0=r0):
                pltpu.sync_copy(rows0, acc_sh.at[pl.ds(r0, CHUNK)])

            if j < 400 // CHUNK:
                cp()
            else:
                pl.when(s < NS - 1)(cp)

        plsc.subcore_barrier()

        sbase = (1 - grow) * E + s * EPT

        def istart(k, buf, sem):
            pltpu.async_copy(ei_ref.at[pl.ds(sbase + k * CHUNK, CHUNK)], buf,
                             sem)

        def iwait(buf, sem):
            pltpu.make_async_copy(ei_ref.at[pl.ds(0, CHUNK)], buf, sem).wait()

        def gstart(lk, buf, sem):
            pltpu.async_copy(
                h_ref.at[gidx_flat.at[pl.ds(lk * CHUNK, CHUNK)]], buf, sem)

        def gwait(buf, sem):
            pltpu.make_async_copy(h_ref.at[pl.ds(0, CHUNK)], buf, sem).wait()

        PH = NCH // 2
        NFULL = (PH // RING) * RING

        def run_phase(ph):
            load_gidx(ph)
            for b in range(RING):
                istart(ph * PH + b, sbufs[b], isems[b])
                gstart(b, rbufs[b], gsems[b])

            def step(lk, b):
                gwait(rbufs[b], gsems[b])
                iwait(sbufs[b], isems[b])
                pltpu.sync_copy(rbufs[b], acc_sh.at[sbufs[b]], add=True)

            def body(kk, carry):
                for b in range(RING):
                    lk = kk * RING + b
                    step(lk, b)

                    @pl.when(lk + RING < PH)
                    def _(b=b, lk=lk):
                        istart(ph * PH + lk + RING, sbufs[b], isems[b])
                        gstart(lk + RING, rbufs[b], gsems[b])

                return carry

            lax.fori_loop(0, PH // RING, body, 0)
            for lk in range(NFULL, PH):
                step(lk, lk % RING)

        run_phase(0)
        run_phase(1)
        plsc.subcore_barrier()

        for j in range(ZROWS // CHUNK):
            r0 = s * ZROWS + j * CHUNK

            def wb(r0=r0):
                pltpu.sync_copy(acc_sh.at[pl.ds(r0, CHUNK)], rows0)
                pltpu.sync_copy(rows0, out_ref.at[pl.ds(coff + r0, CHUNK)])

            if j < 400 // CHUNK:
                wb()
            else:
                pl.when(s < NS - 1)(wb)

    return _agg


_agg_a = _make_agg(False)
_agg_b = _make_agg(True)


def _rscales(deg_blk):
    r_out = lax.rsqrt(deg_blk[:, 0:1] + 1.0)
    r_in = lax.rsqrt(deg_blk[:, 1:2] + 1.0)
    return r_out, r_in


def _mm(x, w_ref, b_ref):
    w = w_ref[0]
    b = b_ref[0]
    return (
        lax.dot_general(x, w, (((1,), (1,)), ((), ())),
                        preferred_element_type=jnp.float32)
        + b
    )


def _tc1_body(deg_ref, x_ref, w_ref, b_ref, hsc_ref, diag_ref):
    chain = pl.program_id(0)
    r_out, r_in = _rscales(deg_ref[...])
    h = _mm(x_ref[...], w_ref, b_ref)
    f = jnp.where(chain == 0, r_out, r_in)
    hsc_ref[...] = h * f
    diag_ref[...] = h * (r_in * r_out)


def _tc2_body(deg_ref, raw_ref, diag_ref, w_ref, b_ref, zsc_ref, diag2_ref):
    chain = pl.program_id(0)
    r_out, r_in = _rscales(deg_ref[...])
    f1 = jnp.where(chain == 0, r_in, r_out)
    a = jnp.maximum(raw_ref[...] * f1 + diag_ref[...], 0.0)
    z = _mm(a, w_ref, b_ref)
    zsc_ref[...] = z * f1
    diag2_ref[...] = z * (r_in * r_out)


def _tc3_body(deg_ref, raw_ref, diag2_ref, mu_ref):
    chain = pl.program_id(0)
    r_out, r_in = _rscales(deg_ref[...])
    f2 = jnp.where(chain == 0, r_out, r_in)
    mu_ref[...] = raw_ref[...] * f2 + diag2_ref[...]


def _row_spec():
    return pl.BlockSpec((RB, D), lambda c, i: (c * NB + i, 0))


def _deg_spec():
    return pl.BlockSpec((RB, 2), lambda c, i: (i, 0))


def _w_spec():
    return pl.BlockSpec((1, D, D), lambda c, i: (c, 0, 0))


def _b_spec():
    return pl.BlockSpec((1, 1, D), lambda c, i: (c, 0, 0))


def _tc1(degT, xs, w1, b1):
    return pl.pallas_call(
        _tc1_body,
        grid=(2, NB),
        in_specs=[_deg_spec(), _row_spec(), _w_spec(), _b_spec()],
        out_specs=[_row_spec(), _row_spec()],
        out_shape=[jax.ShapeDtypeStruct((2 * N, D), jnp.float32)] * 2,
    )(degT, xs, w1, b1)


def _tc2(degT, raw, diag, w4, b4):
    return pl.pallas_call(
        _tc2_body,
        grid=(2, NB),
        in_specs=[_deg_spec(), _row_spec(), _row_spec(), _w_spec(), _b_spec()],
        out_specs=[_row_spec(), _row_spec()],
        out_shape=[jax.ShapeDtypeStruct((2 * N, D), jnp.float32)] * 2,
    )(degT, raw, diag, w4, b4)


def _tc3(degT, raw, diag2):
    return pl.pallas_call(
        _tc3_body,
        grid=(2, NB),
        in_specs=[_deg_spec(), _row_spec(), _row_spec()],
        out_specs=_row_spec(),
        out_shape=jax.ShapeDtypeStruct((2 * N, D), jnp.float32),
    )(degT, raw, diag2)


def kernel(s, t, edge_index, W_sm1, b_sm1, W_sm4, b_sm4, W_sl1, b_sl1,
           W_sl4, b_sl4, W_tm1, b_tm1, W_tm4, b_tm4, W_tl1, b_tl1,
           W_tl4, b_tl4):
    ei_flat = edge_index.reshape(2 * E)
    deg = _deg_kernel(ei_flat)
    degT = deg.reshape(2, N).T

    xs = jnp.concatenate([s, t], axis=0)
    w1 = jnp.stack([W_sm1, W_tm1])
    b1 = jnp.stack([b_sm1, b_tm1])[:, None, :]
    w4 = jnp.stack([W_sm4, W_tm4])
    b4 = jnp.stack([b_sm4, b_tm4])[:, None, :]

    hsc, diag1 = _tc1(degT, xs, w1, b1)
    raw_a = _agg_a(ei_flat, hsc)
    zsc, diag2 = _tc2(degT, raw_a, diag1, w4, b4)
    raw_b = _agg_b(ei_flat, zsc)
    mu = _tc3(degT, raw_b, diag2)
    return mu[:N], mu[N:]
